# baseline (speedup 1.0000x reference)
import jax
import jax.numpy as jnp
from jax import lax
from jax.experimental import pallas as pl
from jax.experimental.pallas import tpu as pltpu
from jax.experimental.pallas import tpu_sc as plsc

_N = 20000
_G = 64
_NE = _N + _G
_LANES = 128
_ROWS = 160
_PAD = _ROWS * _LANES
_NCLS = 81
_FG = 32
_NBG = 96
_NROI = 128
_BG_PID = 5532.0

_NC = 1
_NS = 16
_NW = _NC * _NS
_CHUNK = _PAD // _NW
_NVR = _CHUNK // 16
_OW = 128
_DUMMY = _NROI
_NSTG = 224


def _iou_body(x1_ref, y1_ref, x2_ref, y2_ref, gt_ref, mx_ref, ag_ref):
    x1 = x1_ref[...]
    y1 = y1_ref[...]
    x2 = x2_ref[...]
    y2 = y2_ref[...]
    area_b = (x2 - x1 + 1.0) * (y2 - y1 + 1.0)

    def gt_step(g, carry):
        mx, ag = carry
        gx1 = gt_ref[g, 0]
        gy1 = gt_ref[g, 1]
        gx2 = gt_ref[g, 2]
        gy2 = gt_ref[g, 3]
        area_q = (gx2 - gx1 + 1.0) * (gy2 - gy1 + 1.0)
        iw = jnp.minimum(x2, gx2) - jnp.maximum(x1, gx1) + 1.0
        ih = jnp.minimum(y2, gy2) - jnp.maximum(y1, gy1) + 1.0
        iw = jnp.maximum(iw, 0.0)
        ih = jnp.maximum(ih, 0.0)
        inter = iw * ih
        ua = area_b + area_q - inter
        iou = inter / ua
        upd = iou > mx
        mx = jnp.where(upd, iou, mx)
        ag = jnp.where(upd, g, ag)
        return mx, ag

    mx0 = jnp.full((_ROWS, _LANES), -1.0, jnp.float32)
    ag0 = jnp.zeros((_ROWS, _LANES), jnp.int32)
    mx, ag = lax.fori_loop(0, _G, gt_step, (mx0, ag0))
    mx_ref[...] = mx
    ag_ref[...] = ag.astype(jnp.float32)


def _run_iou(x1, y1, x2, y2, gt):
    return pl.pallas_call(
        _iou_body,
        out_shape=[
            jax.ShapeDtypeStruct((_ROWS, _LANES), jnp.float32),
            jax.ShapeDtypeStruct((_ROWS, _LANES), jnp.float32),
        ],
        in_specs=[
            pl.BlockSpec(memory_space=pltpu.VMEM),
            pl.BlockSpec(memory_space=pltpu.VMEM),
            pl.BlockSpec(memory_space=pltpu.VMEM),
            pl.BlockSpec(memory_space=pltpu.VMEM),
            pl.BlockSpec(memory_space=pltpu.SMEM),
        ],
    )(x1, y1, x2, y2, gt)


def _sc_body(mx_hbm, x1_hbm, y1_hbm, x2_hbm, y2_hbm, ag_hbm, out_hbm,
             mx_v, f0_v, f1_v, f2_v, f3_v, f4_v,
             candfg_v, candbg_v, cnt_v, allc_v, shared, stg_v, slot_v, sem):
    wid = lax.axis_index("s") * _NC + lax.axis_index("c")
    base = wid * _CHUNK
    fv = (f0_v, f1_v, f2_v, f3_v, f4_v)
    cps = [pltpu.async_copy(mx_hbm.at[pl.ds(base, _CHUNK)], mx_v, sem)]
    for c, src in enumerate((x1_hbm, y1_hbm, x2_hbm, y2_hbm, ag_hbm)):
        cps.append(pltpu.async_copy(src.at[pl.ds(base, _CHUNK)], fv[c], sem))
    for cp in cps:
        cp.wait()

    zeros16 = jnp.zeros((16,), jnp.int32)
    for i in range(_FG // 16 + 1):
        candfg_v[pl.ds(16 * i, 16)] = zeros16
    for i in range(_NBG // 16 + 1):
        candbg_v[pl.ds(16 * i, 16)] = zeros16

    lane = lax.iota(jnp.int32, 16)
    carry_fg = jnp.zeros((16,), jnp.int32)
    carry_bg = jnp.zeros((16,), jnp.int32)
    for v in range(_NVR):
        mxv = mx_v[pl.ds(16 * v, 16)]
        lin = base + 16 * v + lane
        loc = 16 * v + lane
        ok = lin < _NE
        mfg = (mxv >= 0.5) & ok
        mbg = (mxv < 0.5) & (mxv >= 0.0) & ok
        rfg = plsc.cumsum(mfg.astype(jnp.int32)) + carry_fg
        rbg = plsc.cumsum(mbg.astype(jnp.int32)) + carry_bg
        plsc.store_scatter(candfg_v, [rfg - 1], loc, mask=mfg & (rfg <= _FG))
        plsc.store_scatter(candbg_v, [rbg - 1], loc, mask=mbg & (rbg <= _NBG))
        carry_fg = carry_fg + plsc.all_reduce_population_count(mfg)
        carry_bg = carry_bg + plsc.all_reduce_population_count(mbg)

    cnt_v[...] = carry_fg * 4096 + carry_bg
    pltpu.sync_copy(cnt_v, shared.at[wid])
    plsc.subcore_barrier()
    pltpu.sync_copy(shared, allc_v)

    widv = jnp.full((16,), wid, jnp.int32)
    fg_off = jnp.zeros((16,), jnp.int32)
    bg_off = jnp.zeros((16,), jnp.int32)
    bg_tot = jnp.zeros((16,), jnp.int32)
    for w in range(_NW):
        rv = allc_v[w]
        fgw = rv // 4096
        bgw = rv % 4096
        before = jnp.full((16,), w, jnp.int32) < widv
        fg_off = fg_off + jnp.where(before, fgw, 0)
        bg_off = bg_off + jnp.where(before, bgw, 0)
        bg_tot = bg_tot + bgw

    def stage(i, loc, slot, valid):
        slot = jnp.where(valid, slot, _DUMMY)
        slot_v[pl.ds(16 * i, 16)] = slot
        row = 16 * i + lane
        for c in range(5):
            val = plsc.load_gather(fv[c], [loc])
            plsc.store_scatter(stg_v, [row, jnp.full((16,), c, jnp.int32)], val)

    for i in range(_FG // 16):
        k = 16 * i + lane
        slot = fg_off + k
        stage(i, candfg_v[pl.ds(16 * i, 16)], slot,
              (slot < _FG) & (k < carry_fg))
    for i in range(_NBG // 16):
        k = 16 * i + lane
        slot = _FG + bg_off + k
        stage(2 + i, candbg_v[pl.ds(16 * i, 16)], slot,
              (slot < _NROI) & (k < carry_bg))
    iswid0 = widv == 0
    for i in range(_NBG // 16):
        slot = _FG + 16 * i + lane
        stage(8 + i, zeros16, slot, (slot >= _FG + bg_tot) & iswid0)

    pltpu.async_copy(stg_v, out_hbm.at[slot_v], sem).wait()


def _run_sc(mx, x1, y1, x2, y2, ag):
    mesh = plsc.VectorSubcoreMesh(core_axis_name="c", subcore_axis_name="s",
                                  num_cores=_NC, num_subcores=_NS)
    f = pl.kernel(
        _sc_body,
        out_type=jax.ShapeDtypeStruct((_NROI + 8, _OW), jnp.float32),
        mesh=mesh,
        compiler_params=pltpu.CompilerParams(needs_layout_passes=False),
        scratch_types=[
            pltpu.VMEM((_CHUNK,), jnp.float32),
            pltpu.VMEM((_CHUNK,), jnp.float32),
            pltpu.VMEM((_CHUNK,), jnp.float32),
            pltpu.VMEM((_CHUNK,), jnp.float32),
            pltpu.VMEM((_CHUNK,), jnp.float32),
            pltpu.VMEM((_CHUNK,), jnp.float32),
            pltpu.VMEM((_FG + 16,), jnp.int32),
            pltpu.VMEM((_NBG + 16,), jnp.int32),
            pltpu.VMEM((16,), jnp.int32),
            pltpu.VMEM((_NW, 16), jnp.int32),
            pltpu.VMEM_SHARED((_NW, 16), jnp.int32),
            pltpu.VMEM((_NSTG, _OW), jnp.float32),
            pltpu.VMEM((_NSTG,), jnp.int32),
            pltpu.SemaphoreType.DMA,
        ],
    )
    return f(mx, x1, y1, x2, y2, ag)


def _tail_body(sel_ref, gt_ref, cls_ref, tgt_ref, inw_ref, outw_ref, pid_ref):
    sel = sel_ref[...]
    ex1 = sel[:, 0:1]
    ey1 = sel[:, 1:2]
    ex2 = sel[:, 2:3]
    ey2 = sel[:, 3:4]
    asg = sel[:, 4:5]

    col8 = lax.broadcasted_iota(jnp.int32, (_NROI, 8), 1)

    def ggt(g, c):
        gfv = g.astype(jnp.float32)
        m = asg == gfv
        vals = (jnp.where(col8 == 0, gt_ref[g, 0], 0.0)
                + jnp.where(col8 == 1, gt_ref[g, 1], 0.0)
                + jnp.where(col8 == 2, gt_ref[g, 2], 0.0)
                + jnp.where(col8 == 3, gt_ref[g, 3], 0.0)
                + jnp.where(col8 == 4, gt_ref[g, 4], 0.0)
                + jnp.where(col8 == 5, gt_ref[g, 5], 0.0))
        return c + jnp.where(m, vals, 0.0)

    gtacc = lax.fori_loop(0, _G, ggt, jnp.zeros((_NROI, 8), jnp.float32))

    gx1 = gtacc[:, 0:1]
    gy1 = gtacc[:, 1:2]
    gx2 = gtacc[:, 2:3]
    gy2 = gtacc[:, 3:4]
    glab = gtacc[:, 4:5]
    gpid = gtacc[:, 5:6]

    ex_w = ex2 - ex1 + 1.0
    ex_h = ey2 - ey1 + 1.0
    ex_cx = ex1 + 0.5 * ex_w
    ex_cy = ey1 + 0.5 * ex_h
    gt_w = gx2 - gx1 + 1.0
    gt_h = gy2 - gy1 + 1.0
    gt_cx = gx1 + 0.5 * gt_w
    gt_cy = gy1 + 0.5 * gt_h
    dx = (gt_cx - ex_cx) / ex_w
    dy = (gt_cy - ex_cy) / ex_h
    dw = jnp.log(gt_w / ex_w)
    dh = jnp.log(gt_h / ex_h)
    dxn = (dx - 0.0) / 0.1
    dyn_ = (dy - 0.0) / 0.1
    dwn = (dw - 0.0) / 0.2
    dhn = (dh - 0.0) / 0.2

    jr = lax.broadcasted_iota(jnp.int32, (_NROI, 1), 0)
    isfg = jr < _FG
    lab = jnp.where(isfg, glab, 0.0)
    clsf = jnp.round(lab)
    pidf = jnp.where(isfg, jnp.round(gpid), _BG_PID)

    cidx = lax.broadcasted_iota(jnp.int32, (_NROI, 4 * _NCLS), 1)
    cls_i = clsf.astype(jnp.int32)
    rel = cidx - 4 * cls_i
    fgm = cls_i > 0
    tvals = (jnp.where(rel == 0, dxn, 0.0) + jnp.where(rel == 1, dyn_, 0.0)
             + jnp.where(rel == 2, dwn, 0.0) + jnp.where(rel == 3, dhn, 0.0))
    tgt_ref[...] = jnp.where(fgm, tvals, 0.0)
    inw = jnp.where(fgm & (rel >= 0) & (rel <= 3), 1.0, 0.0)
    inw_ref[...] = inw
    outw_ref[...] = inw
    cls_ref[...] = clsf
    pid_ref[...] = pidf


def _run_tail(sel, gt):
    return pl.pallas_call(
        _tail_body,
        out_shape=[
            jax.ShapeDtypeStruct((_NROI, 1), jnp.float32),
            jax.ShapeDtypeStruct((_NROI, 4 * _NCLS), jnp.float32),
            jax.ShapeDtypeStruct((_NROI, 4 * _NCLS), jnp.float32),
            jax.ShapeDtypeStruct((_NROI, 4 * _NCLS), jnp.float32),
            jax.ShapeDtypeStruct((_NROI, 1), jnp.float32),
        ],
        in_specs=[
            pl.BlockSpec(memory_space=pltpu.VMEM),
            pl.BlockSpec(memory_space=pltpu.SMEM),
        ],
    )(sel, gt)


@jax.jit
def kernel(all_rois, gt_boxes):
    coords = jnp.concatenate([all_rois[:, 1:5], gt_boxes[:, :4]], axis=0)
    coords = jnp.pad(coords, ((0, _PAD - _NE), (0, 0)))
    x1 = coords[:, 0].reshape(_ROWS, _LANES)
    y1 = coords[:, 1].reshape(_ROWS, _LANES)
    x2 = coords[:, 2].reshape(_ROWS, _LANES)
    y2 = coords[:, 3].reshape(_ROWS, _LANES)
    mx, agf = _run_iou(x1, y1, x2, y2, gt_boxes)
    out = _run_sc(mx.reshape(_PAD), x1.reshape(_PAD), y1.reshape(_PAD),
                  x2.reshape(_PAD), y2.reshape(_PAD), agf.reshape(_PAD))
    sel = out[:_NROI, :8]
    clsf, tgt, inw, outw, pidf = _run_tail(sel, gt_boxes)
    rois = jnp.concatenate([jnp.zeros((_NROI, 1), jnp.float32), sel[:, :4]],
                           axis=1)
    clss = clsf[:, 0].astype(jnp.int32)
    pid = pidf[:, 0].astype(jnp.int32)
    return (rois, clss, tgt, inw, outw, pid)


# speedup vs baseline: 2.9067x; 2.9067x over previous
import jax
import jax.numpy as jnp
from jax import lax
from jax.experimental import pallas as pl
from jax.experimental.pallas import tpu as pltpu
from jax.experimental.pallas import tpu_sc as plsc

_N = 20000
_G = 64
_NE = _N + _G
_LANES = 128
_ROWS = 160
_PAD = _ROWS * _LANES
_NCLS = 81
_FG = 32
_NBG = 96
_NROI = 128
_BG_PID = 5532.0

_NC = 1
_NS = 16
_NW = _NC * _NS
_CHUNK = _PAD // _NW
_NVR = _CHUNK // 16
_OW = 16
_DUMMY = _NROI
_NSTG = 224


def _iou_body(x1_ref, y1_ref, x2_ref, y2_ref, gt_ref, mx_ref, ag_ref):
    x1 = x1_ref[...]
    y1 = y1_ref[...]
    x2 = x2_ref[...]
    y2 = y2_ref[...]
    area_b = (x2 - x1 + 1.0) * (y2 - y1 + 1.0)

    def gt_step(g, carry):
        mx, ag = carry
        gx1 = gt_ref[g, 0]
        gy1 = gt_ref[g, 1]
        gx2 = gt_ref[g, 2]
        gy2 = gt_ref[g, 3]
        area_q = (gx2 - gx1 + 1.0) * (gy2 - gy1 + 1.0)
        iw = jnp.minimum(x2, gx2) - jnp.maximum(x1, gx1) + 1.0
        ih = jnp.minimum(y2, gy2) - jnp.maximum(y1, gy1) + 1.0
        iw = jnp.maximum(iw, 0.0)
        ih = jnp.maximum(ih, 0.0)
        inter = iw * ih
        ua = area_b + area_q - inter
        iou = inter / ua
        upd = iou > mx
        mx = jnp.where(upd, iou, mx)
        ag = jnp.where(upd, g, ag)
        return mx, ag

    mx0 = jnp.full((_ROWS, _LANES), -1.0, jnp.float32)
    ag0 = jnp.zeros((_ROWS, _LANES), jnp.int32)
    mx, ag = lax.fori_loop(0, _G, gt_step, (mx0, ag0))
    mx_ref[...] = mx
    ag_ref[...] = ag.astype(jnp.float32)


def _run_iou(x1, y1, x2, y2, gt):
    return pl.pallas_call(
        _iou_body,
        out_shape=[
            jax.ShapeDtypeStruct((_ROWS, _LANES), jnp.float32),
            jax.ShapeDtypeStruct((_ROWS, _LANES), jnp.float32),
        ],
        in_specs=[
            pl.BlockSpec(memory_space=pltpu.VMEM),
            pl.BlockSpec(memory_space=pltpu.VMEM),
            pl.BlockSpec(memory_space=pltpu.VMEM),
            pl.BlockSpec(memory_space=pltpu.VMEM),
            pl.BlockSpec(memory_space=pltpu.SMEM),
        ],
    )(x1, y1, x2, y2, gt)


def _sc_body(mx_hbm, x1_hbm, y1_hbm, x2_hbm, y2_hbm, ag_hbm, out_hbm,
             mx_v, f0_v, f1_v, f2_v, f3_v, f4_v,
             candfg_v, candbg_v, cnt_v, allc_v, shared, stg_v, slot_v, sem):
    wid = lax.axis_index("s") * _NC + lax.axis_index("c")
    base = wid * _CHUNK
    fv = (f0_v, f1_v, f2_v, f3_v, f4_v)
    cps = [pltpu.async_copy(mx_hbm.at[pl.ds(base, _CHUNK)], mx_v, sem)]
    for c, src in enumerate((x1_hbm, y1_hbm, x2_hbm, y2_hbm, ag_hbm)):
        cps.append(pltpu.async_copy(src.at[pl.ds(base, _CHUNK)], fv[c], sem))
    for cp in cps:
        cp.wait()

    zeros16 = jnp.zeros((16,), jnp.int32)
    for i in range(_FG // 16 + 1):
        candfg_v[pl.ds(16 * i, 16)] = zeros16
    for i in range(_NBG // 16 + 1):
        candbg_v[pl.ds(16 * i, 16)] = zeros16

    lane = lax.iota(jnp.int32, 16)
    carry_fg = jnp.zeros((16,), jnp.int32)
    carry_bg = jnp.zeros((16,), jnp.int32)
    for v in range(_NVR):
        mxv = mx_v[pl.ds(16 * v, 16)]
        lin = base + 16 * v + lane
        loc = 16 * v + lane
        ok = lin < _NE
        mfg = (mxv >= 0.5) & ok
        mbg = (mxv < 0.5) & (mxv >= 0.0) & ok
        rfg = plsc.cumsum(mfg.astype(jnp.int32)) + carry_fg
        rbg = plsc.cumsum(mbg.astype(jnp.int32)) + carry_bg
        plsc.store_scatter(candfg_v, [rfg - 1], loc, mask=mfg & (rfg <= _FG))
        plsc.store_scatter(candbg_v, [rbg - 1], loc, mask=mbg & (rbg <= _NBG))
        carry_fg = carry_fg + plsc.all_reduce_population_count(mfg)
        carry_bg = carry_bg + plsc.all_reduce_population_count(mbg)

    cnt_v[...] = carry_fg * 4096 + carry_bg
    pltpu.sync_copy(cnt_v, shared.at[wid])
    plsc.subcore_barrier()
    pltpu.sync_copy(shared, allc_v)

    widv = jnp.full((16,), wid, jnp.int32)
    fg_off = jnp.zeros((16,), jnp.int32)
    bg_off = jnp.zeros((16,), jnp.int32)
    bg_tot = jnp.zeros((16,), jnp.int32)
    for w in range(_NW):
        rv = allc_v[w]
        fgw = rv // 4096
        bgw = rv % 4096
        before = jnp.full((16,), w, jnp.int32) < widv
        fg_off = fg_off + jnp.where(before, fgw, 0)
        bg_off = bg_off + jnp.where(before, bgw, 0)
        bg_tot = bg_tot + bgw

    def stage(i, loc, slot, valid):
        slot = jnp.where(valid, slot, _DUMMY)
        slot_v[pl.ds(16 * i, 16)] = slot
        row = 16 * i + lane
        for c in range(5):
            val = plsc.load_gather(fv[c], [loc])
            plsc.store_scatter(stg_v, [row, jnp.full((16,), c, jnp.int32)], val)

    for i in range(_FG // 16):
        k = 16 * i + lane
        slot = fg_off + k
        stage(i, candfg_v[pl.ds(16 * i, 16)], slot,
              (slot < _FG) & (k < carry_fg))
    for i in range(_NBG // 16):
        k = 16 * i + lane
        slot = _FG + bg_off + k
        stage(2 + i, candbg_v[pl.ds(16 * i, 16)], slot,
              (slot < _NROI) & (k < carry_bg))
    iswid0 = widv == 0
    for i in range(_NBG // 16):
        slot = _FG + 16 * i + lane
        stage(8 + i, zeros16, slot, (slot >= _FG + bg_tot) & iswid0)

    pltpu.async_copy(stg_v, out_hbm.at[slot_v], sem).wait()


def _run_sc(mx, x1, y1, x2, y2, ag):
    mesh = plsc.VectorSubcoreMesh(core_axis_name="c", subcore_axis_name="s",
                                  num_cores=_NC, num_subcores=_NS)
    f = pl.kernel(
        _sc_body,
        out_type=jax.ShapeDtypeStruct((_NROI + 8, _OW), jnp.float32),
        mesh=mesh,
        compiler_params=pltpu.CompilerParams(needs_layout_passes=False,
                                             use_tc_tiling_on_sc=False),
        scratch_types=[
            pltpu.VMEM((_CHUNK,), jnp.float32),
            pltpu.VMEM((_CHUNK,), jnp.float32),
            pltpu.VMEM((_CHUNK,), jnp.float32),
            pltpu.VMEM((_CHUNK,), jnp.float32),
            pltpu.VMEM((_CHUNK,), jnp.float32),
            pltpu.VMEM((_CHUNK,), jnp.float32),
            pltpu.VMEM((_FG + 16,), jnp.int32),
            pltpu.VMEM((_NBG + 16,), jnp.int32),
            pltpu.VMEM((16,), jnp.int32),
            pltpu.VMEM((_NW, 16), jnp.int32),
            pltpu.VMEM_SHARED((_NW, 16), jnp.int32),
            pltpu.VMEM((_NSTG, _OW), jnp.float32),
            pltpu.VMEM((_NSTG,), jnp.int32),
            pltpu.SemaphoreType.DMA,
        ],
    )
    return f(mx, x1, y1, x2, y2, ag)


def _tail_body(sel_ref, gt_ref, cls_ref, tgt_ref, inw_ref, outw_ref, pid_ref):
    sel = sel_ref[...]
    ex1 = sel[:, 0:1]
    ey1 = sel[:, 1:2]
    ex2 = sel[:, 2:3]
    ey2 = sel[:, 3:4]
    asg = sel[:, 4:5]

    col8 = lax.broadcasted_iota(jnp.int32, (_NROI, 8), 1)

    def ggt(g, c):
        gfv = g.astype(jnp.float32)
        m = asg == gfv
        vals = (jnp.where(col8 == 0, gt_ref[g, 0], 0.0)
                + jnp.where(col8 == 1, gt_ref[g, 1], 0.0)
                + jnp.where(col8 == 2, gt_ref[g, 2], 0.0)
                + jnp.where(col8 == 3, gt_ref[g, 3], 0.0)
                + jnp.where(col8 == 4, gt_ref[g, 4], 0.0)
                + jnp.where(col8 == 5, gt_ref[g, 5], 0.0))
        return c + jnp.where(m, vals, 0.0)

    gtacc = lax.fori_loop(0, _G, ggt, jnp.zeros((_NROI, 8), jnp.float32))

    gx1 = gtacc[:, 0:1]
    gy1 = gtacc[:, 1:2]
    gx2 = gtacc[:, 2:3]
    gy2 = gtacc[:, 3:4]
    glab = gtacc[:, 4:5]
    gpid = gtacc[:, 5:6]

    ex_w = ex2 - ex1 + 1.0
    ex_h = ey2 - ey1 + 1.0
    ex_cx = ex1 + 0.5 * ex_w
    ex_cy = ey1 + 0.5 * ex_h
    gt_w = gx2 - gx1 + 1.0
    gt_h = gy2 - gy1 + 1.0
    gt_cx = gx1 + 0.5 * gt_w
    gt_cy = gy1 + 0.5 * gt_h
    dx = (gt_cx - ex_cx) / ex_w
    dy = (gt_cy - ex_cy) / ex_h
    dw = jnp.log(gt_w / ex_w)
    dh = jnp.log(gt_h / ex_h)
    dxn = (dx - 0.0) / 0.1
    dyn_ = (dy - 0.0) / 0.1
    dwn = (dw - 0.0) / 0.2
    dhn = (dh - 0.0) / 0.2

    jr = lax.broadcasted_iota(jnp.int32, (_NROI, 1), 0)
    isfg = jr < _FG
    lab = jnp.where(isfg, glab, 0.0)
    clsf = jnp.round(lab)
    pidf = jnp.where(isfg, jnp.round(gpid), _BG_PID)

    cidx = lax.broadcasted_iota(jnp.int32, (_NROI, 4 * _NCLS), 1)
    cls_i = clsf.astype(jnp.int32)
    rel = cidx - 4 * cls_i
    fgm = cls_i > 0
    tvals = (jnp.where(rel == 0, dxn, 0.0) + jnp.where(rel == 1, dyn_, 0.0)
             + jnp.where(rel == 2, dwn, 0.0) + jnp.where(rel == 3, dhn, 0.0))
    tgt_ref[...] = jnp.where(fgm, tvals, 0.0)
    inw = jnp.where(fgm & (rel >= 0) & (rel <= 3), 1.0, 0.0)
    inw_ref[...] = inw
    outw_ref[...] = inw
    cls_ref[...] = clsf
    pid_ref[...] = pidf


def _run_tail(sel, gt):
    return pl.pallas_call(
        _tail_body,
        out_shape=[
            jax.ShapeDtypeStruct((_NROI, 1), jnp.float32),
            jax.ShapeDtypeStruct((_NROI, 4 * _NCLS), jnp.float32),
            jax.ShapeDtypeStruct((_NROI, 4 * _NCLS), jnp.float32),
            jax.ShapeDtypeStruct((_NROI, 4 * _NCLS), jnp.float32),
            jax.ShapeDtypeStruct((_NROI, 1), jnp.float32),
        ],
        in_specs=[
            pl.BlockSpec(memory_space=pltpu.VMEM),
            pl.BlockSpec(memory_space=pltpu.SMEM),
        ],
    )(sel, gt)


@jax.jit
def kernel(all_rois, gt_boxes):
    coords = jnp.concatenate([all_rois[:, 1:5], gt_boxes[:, :4]], axis=0)
    coords = jnp.pad(coords, ((0, _PAD - _NE), (0, 0)))
    x1 = coords[:, 0].reshape(_ROWS, _LANES)
    y1 = coords[:, 1].reshape(_ROWS, _LANES)
    x2 = coords[:, 2].reshape(_ROWS, _LANES)
    y2 = coords[:, 3].reshape(_ROWS, _LANES)
    mx, agf = _run_iou(x1, y1, x2, y2, gt_boxes)
    out = _run_sc(mx.reshape(_PAD), x1.reshape(_PAD), y1.reshape(_PAD),
                  x2.reshape(_PAD), y2.reshape(_PAD), agf.reshape(_PAD))
    sel = out[:_NROI, :8]
    clsf, tgt, inw, outw, pidf = _run_tail(sel, gt_boxes)
    rois = jnp.concatenate([jnp.zeros((_NROI, 1), jnp.float32), sel[:, :4]],
                           axis=1)
    clss = clsf[:, 0].astype(jnp.int32)
    pid = pidf[:, 0].astype(jnp.int32)
    return (rois, clss, tgt, inw, outw, pid)


# speedup vs baseline: 3.1377x; 1.0795x over previous
import jax
import jax.numpy as jnp
from jax import lax
from jax.experimental import pallas as pl
from jax.experimental.pallas import tpu as pltpu
from jax.experimental.pallas import tpu_sc as plsc

_N = 20000
_G = 64
_NE = _N + _G
_LANES = 128
_ROWS = 160
_PAD = _ROWS * _LANES
_NCLS = 81
_FG = 32
_NBG = 96
_NROI = 128
_BG_PID = 5532.0

_NC = 1
_NS = 16
_NW = _NC * _NS
_CHUNK = _PAD // _NW
_NVR = _CHUNK // 16
_OW = 16
_DUMMY = _NROI
_NSTG = 224


def _iou_body(x1_ref, y1_ref, x2_ref, y2_ref, gt_ref, mx_ref, ag_ref):
    x1 = x1_ref[...]
    y1 = y1_ref[...]
    x2 = x2_ref[...]
    y2 = y2_ref[...]
    area_b = (x2 - x1 + 1.0) * (y2 - y1 + 1.0)

    def gt_step(g, carry):
        mx, ag = carry
        gx1 = gt_ref[g, 0]
        gy1 = gt_ref[g, 1]
        gx2 = gt_ref[g, 2]
        gy2 = gt_ref[g, 3]
        area_q = (gx2 - gx1 + 1.0) * (gy2 - gy1 + 1.0)
        iw = jnp.minimum(x2, gx2) - jnp.maximum(x1, gx1) + 1.0
        ih = jnp.minimum(y2, gy2) - jnp.maximum(y1, gy1) + 1.0
        iw = jnp.maximum(iw, 0.0)
        ih = jnp.maximum(ih, 0.0)
        inter = iw * ih
        ua = area_b + area_q - inter
        iou = inter / ua
        upd = iou > mx
        mx = jnp.where(upd, iou, mx)
        ag = jnp.where(upd, g, ag)
        return mx, ag

    mx0 = jnp.full((_ROWS, _LANES), -1.0, jnp.float32)
    ag0 = jnp.zeros((_ROWS, _LANES), jnp.int32)
    mx, ag = lax.fori_loop(0, _G, gt_step, (mx0, ag0), unroll=8)
    mx_ref[...] = mx
    ag_ref[...] = ag.astype(jnp.float32)


def _run_iou(x1, y1, x2, y2, gt):
    return pl.pallas_call(
        _iou_body,
        out_shape=[
            jax.ShapeDtypeStruct((_ROWS, _LANES), jnp.float32),
            jax.ShapeDtypeStruct((_ROWS, _LANES), jnp.float32),
        ],
        in_specs=[
            pl.BlockSpec(memory_space=pltpu.VMEM),
            pl.BlockSpec(memory_space=pltpu.VMEM),
            pl.BlockSpec(memory_space=pltpu.VMEM),
            pl.BlockSpec(memory_space=pltpu.VMEM),
            pl.BlockSpec(memory_space=pltpu.SMEM),
        ],
    )(x1, y1, x2, y2, gt)


def _sc_body(mx_hbm, x1_hbm, y1_hbm, x2_hbm, y2_hbm, ag_hbm, out_hbm,
             mx_v, f0_v, f1_v, f2_v, f3_v, f4_v,
             candfg_v, candbg_v, cnt_v, allc_v, shared, stg_v, slot_v, sem):
    wid = lax.axis_index("s") * _NC + lax.axis_index("c")
    base = wid * _CHUNK
    fv = (f0_v, f1_v, f2_v, f3_v, f4_v)
    cps = [pltpu.async_copy(mx_hbm.at[pl.ds(base, _CHUNK)], mx_v, sem)]
    for c, src in enumerate((x1_hbm, y1_hbm, x2_hbm, y2_hbm, ag_hbm)):
        cps.append(pltpu.async_copy(src.at[pl.ds(base, _CHUNK)], fv[c], sem))
    for cp in cps:
        cp.wait()

    zeros16 = jnp.zeros((16,), jnp.int32)
    for i in range(_FG // 16 + 1):
        candfg_v[pl.ds(16 * i, 16)] = zeros16
    for i in range(_NBG // 16 + 1):
        candbg_v[pl.ds(16 * i, 16)] = zeros16

    lane = lax.iota(jnp.int32, 16)
    carry_fg = jnp.zeros((16,), jnp.int32)
    carry_bg = jnp.zeros((16,), jnp.int32)
    for v in range(_NVR):
        mxv = mx_v[pl.ds(16 * v, 16)]
        lin = base + 16 * v + lane
        loc = 16 * v + lane
        ok = lin < _NE
        mfg = (mxv >= 0.5) & ok
        mbg = (mxv < 0.5) & (mxv >= 0.0) & ok
        rfg = plsc.cumsum(mfg.astype(jnp.int32)) + carry_fg
        rbg = plsc.cumsum(mbg.astype(jnp.int32)) + carry_bg
        plsc.store_scatter(candfg_v, [rfg - 1], loc, mask=mfg & (rfg <= _FG))
        plsc.store_scatter(candbg_v, [rbg - 1], loc, mask=mbg & (rbg <= _NBG))
        carry_fg = carry_fg + plsc.all_reduce_population_count(mfg)
        carry_bg = carry_bg + plsc.all_reduce_population_count(mbg)

    cnt_v[...] = carry_fg * 4096 + carry_bg
    pltpu.sync_copy(cnt_v, shared.at[wid])
    plsc.subcore_barrier()
    pltpu.sync_copy(shared, allc_v)

    widv = jnp.full((16,), wid, jnp.int32)
    fg_off = jnp.zeros((16,), jnp.int32)
    bg_off = jnp.zeros((16,), jnp.int32)
    bg_tot = jnp.zeros((16,), jnp.int32)
    for w in range(_NW):
        rv = allc_v[w]
        fgw = rv // 4096
        bgw = rv % 4096
        before = jnp.full((16,), w, jnp.int32) < widv
        fg_off = fg_off + jnp.where(before, fgw, 0)
        bg_off = bg_off + jnp.where(before, bgw, 0)
        bg_tot = bg_tot + bgw

    def stage(i, loc, slot, valid):
        slot = jnp.where(valid, slot, _DUMMY)
        slot_v[pl.ds(16 * i, 16)] = slot
        row = 16 * i + lane
        for c in range(5):
            val = plsc.load_gather(fv[c], [loc])
            plsc.store_scatter(stg_v, [row, jnp.full((16,), c, jnp.int32)], val)

    for i in range(_FG // 16):
        k = 16 * i + lane
        slot = fg_off + k
        stage(i, candfg_v[pl.ds(16 * i, 16)], slot,
              (slot < _FG) & (k < carry_fg))
    for i in range(_NBG // 16):
        k = 16 * i + lane
        slot = _FG + bg_off + k
        stage(2 + i, candbg_v[pl.ds(16 * i, 16)], slot,
              (slot < _NROI) & (k < carry_bg))
    iswid0 = widv == 0
    for i in range(_NBG // 16):
        slot = _FG + 16 * i + lane
        stage(8 + i, zeros16, slot, (slot >= _FG + bg_tot) & iswid0)

    pltpu.async_copy(stg_v, out_hbm.at[slot_v], sem).wait()


def _run_sc(mx, x1, y1, x2, y2, ag):
    mesh = plsc.VectorSubcoreMesh(core_axis_name="c", subcore_axis_name="s",
                                  num_cores=_NC, num_subcores=_NS)
    f = pl.kernel(
        _sc_body,
        out_type=jax.ShapeDtypeStruct((_NROI + 8, _OW), jnp.float32),
        mesh=mesh,
        compiler_params=pltpu.CompilerParams(needs_layout_passes=False,
                                             use_tc_tiling_on_sc=False),
        scratch_types=[
            pltpu.VMEM((_CHUNK,), jnp.float32),
            pltpu.VMEM((_CHUNK,), jnp.float32),
            pltpu.VMEM((_CHUNK,), jnp.float32),
            pltpu.VMEM((_CHUNK,), jnp.float32),
            pltpu.VMEM((_CHUNK,), jnp.float32),
            pltpu.VMEM((_CHUNK,), jnp.float32),
            pltpu.VMEM((_FG + 16,), jnp.int32),
            pltpu.VMEM((_NBG + 16,), jnp.int32),
            pltpu.VMEM((16,), jnp.int32),
            pltpu.VMEM((_NW, 16), jnp.int32),
            pltpu.VMEM_SHARED((_NW, 16), jnp.int32),
            pltpu.VMEM((_NSTG, _OW), jnp.float32),
            pltpu.VMEM((_NSTG,), jnp.int32),
            pltpu.SemaphoreType.DMA,
        ],
    )
    return f(mx, x1, y1, x2, y2, ag)


def _tail_body(sel_ref, gt_ref, cls_ref, tgt_ref, inw_ref, outw_ref, pid_ref):
    sel = sel_ref[...]
    ex1 = sel[:, 0:1]
    ey1 = sel[:, 1:2]
    ex2 = sel[:, 2:3]
    ey2 = sel[:, 3:4]
    asg = sel[:, 4:5]

    col8 = lax.broadcasted_iota(jnp.int32, (_NROI, 8), 1)

    def ggt(g, c):
        gfv = g.astype(jnp.float32)
        m = asg == gfv
        vals = (jnp.where(col8 == 0, gt_ref[g, 0], 0.0)
                + jnp.where(col8 == 1, gt_ref[g, 1], 0.0)
                + jnp.where(col8 == 2, gt_ref[g, 2], 0.0)
                + jnp.where(col8 == 3, gt_ref[g, 3], 0.0)
                + jnp.where(col8 == 4, gt_ref[g, 4], 0.0)
                + jnp.where(col8 == 5, gt_ref[g, 5], 0.0))
        return c + jnp.where(m, vals, 0.0)

    gtacc = lax.fori_loop(0, _G, ggt, jnp.zeros((_NROI, 8), jnp.float32),
                          unroll=8)

    gx1 = gtacc[:, 0:1]
    gy1 = gtacc[:, 1:2]
    gx2 = gtacc[:, 2:3]
    gy2 = gtacc[:, 3:4]
    glab = gtacc[:, 4:5]
    gpid = gtacc[:, 5:6]

    ex_w = ex2 - ex1 + 1.0
    ex_h = ey2 - ey1 + 1.0
    ex_cx = ex1 + 0.5 * ex_w
    ex_cy = ey1 + 0.5 * ex_h
    gt_w = gx2 - gx1 + 1.0
    gt_h = gy2 - gy1 + 1.0
    gt_cx = gx1 + 0.5 * gt_w
    gt_cy = gy1 + 0.5 * gt_h
    dx = (gt_cx - ex_cx) / ex_w
    dy = (gt_cy - ex_cy) / ex_h
    dw = jnp.log(gt_w / ex_w)
    dh = jnp.log(gt_h / ex_h)
    dxn = (dx - 0.0) / 0.1
    dyn_ = (dy - 0.0) / 0.1
    dwn = (dw - 0.0) / 0.2
    dhn = (dh - 0.0) / 0.2

    jr = lax.broadcasted_iota(jnp.int32, (_NROI, 1), 0)
    isfg = jr < _FG
    lab = jnp.where(isfg, glab, 0.0)
    clsf = jnp.round(lab)
    pidf = jnp.where(isfg, jnp.round(gpid), _BG_PID)

    cidx = lax.broadcasted_iota(jnp.int32, (_NROI, 4 * _NCLS), 1)
    cls_i = clsf.astype(jnp.int32)
    rel = cidx - 4 * cls_i
    fgm = cls_i > 0
    tvals = (jnp.where(rel == 0, dxn, 0.0) + jnp.where(rel == 1, dyn_, 0.0)
             + jnp.where(rel == 2, dwn, 0.0) + jnp.where(rel == 3, dhn, 0.0))
    tgt_ref[...] = jnp.where(fgm, tvals, 0.0)
    inw = jnp.where(fgm & (rel >= 0) & (rel <= 3), 1.0, 0.0)
    inw_ref[...] = inw
    outw_ref[...] = inw
    cls_ref[...] = clsf
    pid_ref[...] = pidf


def _run_tail(sel, gt):
    return pl.pallas_call(
        _tail_body,
        out_shape=[
            jax.ShapeDtypeStruct((_NROI, 1), jnp.float32),
            jax.ShapeDtypeStruct((_NROI, 4 * _NCLS), jnp.float32),
            jax.ShapeDtypeStruct((_NROI, 4 * _NCLS), jnp.float32),
            jax.ShapeDtypeStruct((_NROI, 4 * _NCLS), jnp.float32),
            jax.ShapeDtypeStruct((_NROI, 1), jnp.float32),
        ],
        in_specs=[
            pl.BlockSpec(memory_space=pltpu.VMEM),
            pl.BlockSpec(memory_space=pltpu.SMEM),
        ],
    )(sel, gt)


@jax.jit
def kernel(all_rois, gt_boxes):
    coords = jnp.concatenate([all_rois[:, 1:5], gt_boxes[:, :4]], axis=0)
    coords = jnp.pad(coords, ((0, _PAD - _NE), (0, 0)))
    x1 = coords[:, 0].reshape(_ROWS, _LANES)
    y1 = coords[:, 1].reshape(_ROWS, _LANES)
    x2 = coords[:, 2].reshape(_ROWS, _LANES)
    y2 = coords[:, 3].reshape(_ROWS, _LANES)
    mx, agf = _run_iou(x1, y1, x2, y2, gt_boxes)
    out = _run_sc(mx.reshape(_PAD), x1.reshape(_PAD), y1.reshape(_PAD),
                  x2.reshape(_PAD), y2.reshape(_PAD), agf.reshape(_PAD))
    sel = out[:_NROI, :8]
    clsf, tgt, inw, outw, pidf = _run_tail(sel, gt_boxes)
    rois = jnp.concatenate([jnp.zeros((_NROI, 1), jnp.float32), sel[:, :4]],
                           axis=1)
    clss = clsf[:, 0].astype(jnp.int32)
    pid = pidf[:, 0].astype(jnp.int32)
    return (rois, clss, tgt, inw, outw, pid)


# speedup vs baseline: 3.1404x; 1.0009x over previous
import jax
import jax.numpy as jnp
from jax import lax
from jax.experimental import pallas as pl
from jax.experimental.pallas import tpu as pltpu
from jax.experimental.pallas import tpu_sc as plsc

_N = 20000
_G = 64
_NE = _N + _G
_LANES = 128
_ROWS = 160
_PAD = _ROWS * _LANES
_NCLS = 81
_FG = 32
_NBG = 96
_NROI = 128
_BG_PID = 5532.0

_NC = 1
_NS = 16
_NW = _NC * _NS
_CHUNK = _PAD // _NW
_NVR = _CHUNK // 16
_OW = 16
_DUMMY = _NROI
_NSTG = 224


def _iou_body(x1_ref, y1_ref, x2_ref, y2_ref, gt_ref, mx_ref, ag_ref):
    x1 = x1_ref[...]
    y1 = y1_ref[...]
    x2 = x2_ref[...]
    y2 = y2_ref[...]
    area_b = (x2 - x1 + 1.0) * (y2 - y1 + 1.0)

    def gt_step(g, carry):
        mx, ag = carry
        gx1 = gt_ref[g, 0]
        gy1 = gt_ref[g, 1]
        gx2 = gt_ref[g, 2]
        gy2 = gt_ref[g, 3]
        area_q = (gx2 - gx1 + 1.0) * (gy2 - gy1 + 1.0)
        iw = jnp.minimum(x2, gx2) - jnp.maximum(x1, gx1) + 1.0
        ih = jnp.minimum(y2, gy2) - jnp.maximum(y1, gy1) + 1.0
        iw = jnp.maximum(iw, 0.0)
        ih = jnp.maximum(ih, 0.0)
        inter = iw * ih
        ua = area_b + area_q - inter
        iou = inter / ua
        upd = iou > mx
        mx = jnp.where(upd, iou, mx)
        ag = jnp.where(upd, g, ag)
        return mx, ag

    mx0 = jnp.full((_ROWS, _LANES), -1.0, jnp.float32)
    ag0 = jnp.zeros((_ROWS, _LANES), jnp.int32)
    mx, ag = lax.fori_loop(0, _G, gt_step, (mx0, ag0), unroll=16)
    mx_ref[...] = mx
    ag_ref[...] = ag.astype(jnp.float32)


def _run_iou(x1, y1, x2, y2, gt):
    return pl.pallas_call(
        _iou_body,
        out_shape=[
            jax.ShapeDtypeStruct((_ROWS, _LANES), jnp.float32),
            jax.ShapeDtypeStruct((_ROWS, _LANES), jnp.float32),
        ],
        in_specs=[
            pl.BlockSpec(memory_space=pltpu.VMEM),
            pl.BlockSpec(memory_space=pltpu.VMEM),
            pl.BlockSpec(memory_space=pltpu.VMEM),
            pl.BlockSpec(memory_space=pltpu.VMEM),
            pl.BlockSpec(memory_space=pltpu.SMEM),
        ],
    )(x1, y1, x2, y2, gt)


def _sc_body(mx_hbm, x1_hbm, y1_hbm, x2_hbm, y2_hbm, ag_hbm, out_hbm,
             mx_v, f0_v, f1_v, f2_v, f3_v, f4_v,
             candfg_v, candbg_v, cnt_v, allc_v, shared, stg_v, slot_v, sem):
    wid = lax.axis_index("s") * _NC + lax.axis_index("c")
    base = wid * _CHUNK
    fv = (f0_v, f1_v, f2_v, f3_v, f4_v)
    cps = [pltpu.async_copy(mx_hbm.at[pl.ds(base, _CHUNK)], mx_v, sem)]
    for c, src in enumerate((x1_hbm, y1_hbm, x2_hbm, y2_hbm, ag_hbm)):
        cps.append(pltpu.async_copy(src.at[pl.ds(base, _CHUNK)], fv[c], sem))
    for cp in cps:
        cp.wait()

    zeros16 = jnp.zeros((16,), jnp.int32)
    for i in range(_FG // 16 + 1):
        candfg_v[pl.ds(16 * i, 16)] = zeros16
    for i in range(_NBG // 16 + 1):
        candbg_v[pl.ds(16 * i, 16)] = zeros16

    lane = lax.iota(jnp.int32, 16)
    carry_fg = jnp.zeros((16,), jnp.int32)
    carry_bg = jnp.zeros((16,), jnp.int32)
    for v in range(_NVR):
        mxv = mx_v[pl.ds(16 * v, 16)]
        lin = base + 16 * v + lane
        loc = 16 * v + lane
        ok = lin < _NE
        mfg = (mxv >= 0.5) & ok
        mbg = (mxv < 0.5) & (mxv >= 0.0) & ok
        rfg = plsc.cumsum(mfg.astype(jnp.int32)) + carry_fg
        rbg = plsc.cumsum(mbg.astype(jnp.int32)) + carry_bg
        plsc.store_scatter(candfg_v, [rfg - 1], loc, mask=mfg & (rfg <= _FG))
        plsc.store_scatter(candbg_v, [rbg - 1], loc, mask=mbg & (rbg <= _NBG))
        carry_fg = carry_fg + plsc.all_reduce_population_count(mfg)
        carry_bg = carry_bg + plsc.all_reduce_population_count(mbg)

    cnt_v[...] = carry_fg * 4096 + carry_bg
    pltpu.sync_copy(cnt_v, shared.at[wid])
    plsc.subcore_barrier()
    pltpu.sync_copy(shared, allc_v)

    widv = jnp.full((16,), wid, jnp.int32)
    fg_off = jnp.zeros((16,), jnp.int32)
    bg_off = jnp.zeros((16,), jnp.int32)
    bg_tot = jnp.zeros((16,), jnp.int32)
    for w in range(_NW):
        rv = allc_v[w]
        fgw = rv // 4096
        bgw = rv % 4096
        before = jnp.full((16,), w, jnp.int32) < widv
        fg_off = fg_off + jnp.where(before, fgw, 0)
        bg_off = bg_off + jnp.where(before, bgw, 0)
        bg_tot = bg_tot + bgw

    def stage(i, loc, slot, valid):
        slot = jnp.where(valid, slot, _DUMMY)
        slot_v[pl.ds(16 * i, 16)] = slot
        row = 16 * i + lane
        for c in range(5):
            val = plsc.load_gather(fv[c], [loc])
            plsc.store_scatter(stg_v, [row, jnp.full((16,), c, jnp.int32)], val)

    for i in range(_FG // 16):
        k = 16 * i + lane
        slot = fg_off + k
        stage(i, candfg_v[pl.ds(16 * i, 16)], slot,
              (slot < _FG) & (k < carry_fg))
    for i in range(_NBG // 16):
        k = 16 * i + lane
        slot = _FG + bg_off + k
        stage(2 + i, candbg_v[pl.ds(16 * i, 16)], slot,
              (slot < _NROI) & (k < carry_bg))
    iswid0 = widv == 0
    for i in range(_NBG // 16):
        slot = _FG + 16 * i + lane
        stage(8 + i, zeros16, slot, (slot >= _FG + bg_tot) & iswid0)

    pltpu.async_copy(stg_v, out_hbm.at[slot_v], sem).wait()


def _run_sc(mx, x1, y1, x2, y2, ag):
    mesh = plsc.VectorSubcoreMesh(core_axis_name="c", subcore_axis_name="s",
                                  num_cores=_NC, num_subcores=_NS)
    f = pl.kernel(
        _sc_body,
        out_type=jax.ShapeDtypeStruct((_NROI + 8, _OW), jnp.float32),
        mesh=mesh,
        compiler_params=pltpu.CompilerParams(needs_layout_passes=False,
                                             use_tc_tiling_on_sc=False),
        scratch_types=[
            pltpu.VMEM((_CHUNK,), jnp.float32),
            pltpu.VMEM((_CHUNK,), jnp.float32),
            pltpu.VMEM((_CHUNK,), jnp.float32),
            pltpu.VMEM((_CHUNK,), jnp.float32),
            pltpu.VMEM((_CHUNK,), jnp.float32),
            pltpu.VMEM((_CHUNK,), jnp.float32),
            pltpu.VMEM((_FG + 16,), jnp.int32),
            pltpu.VMEM((_NBG + 16,), jnp.int32),
            pltpu.VMEM((16,), jnp.int32),
            pltpu.VMEM((_NW, 16), jnp.int32),
            pltpu.VMEM_SHARED((_NW, 16), jnp.int32),
            pltpu.VMEM((_NSTG, _OW), jnp.float32),
            pltpu.VMEM((_NSTG,), jnp.int32),
            pltpu.SemaphoreType.DMA,
        ],
    )
    return f(mx, x1, y1, x2, y2, ag)


def _tail_body(sel_ref, gt_ref, cls_ref, tgt_ref, inw_ref, outw_ref, pid_ref):
    sel = sel_ref[...]
    ex1 = sel[:, 0:1]
    ey1 = sel[:, 1:2]
    ex2 = sel[:, 2:3]
    ey2 = sel[:, 3:4]
    asg = sel[:, 4:5]

    col8 = lax.broadcasted_iota(jnp.int32, (_NROI, 8), 1)

    def ggt(g, c):
        gfv = g.astype(jnp.float32)
        m = asg == gfv
        vals = (jnp.where(col8 == 0, gt_ref[g, 0], 0.0)
                + jnp.where(col8 == 1, gt_ref[g, 1], 0.0)
                + jnp.where(col8 == 2, gt_ref[g, 2], 0.0)
                + jnp.where(col8 == 3, gt_ref[g, 3], 0.0)
                + jnp.where(col8 == 4, gt_ref[g, 4], 0.0)
                + jnp.where(col8 == 5, gt_ref[g, 5], 0.0))
        return c + jnp.where(m, vals, 0.0)

    gtacc = lax.fori_loop(0, _G, ggt, jnp.zeros((_NROI, 8), jnp.float32),
                          unroll=8)

    gx1 = gtacc[:, 0:1]
    gy1 = gtacc[:, 1:2]
    gx2 = gtacc[:, 2:3]
    gy2 = gtacc[:, 3:4]
    glab = gtacc[:, 4:5]
    gpid = gtacc[:, 5:6]

    ex_w = ex2 - ex1 + 1.0
    ex_h = ey2 - ey1 + 1.0
    ex_cx = ex1 + 0.5 * ex_w
    ex_cy = ey1 + 0.5 * ex_h
    gt_w = gx2 - gx1 + 1.0
    gt_h = gy2 - gy1 + 1.0
    gt_cx = gx1 + 0.5 * gt_w
    gt_cy = gy1 + 0.5 * gt_h
    dx = (gt_cx - ex_cx) / ex_w
    dy = (gt_cy - ex_cy) / ex_h
    dw = jnp.log(gt_w / ex_w)
    dh = jnp.log(gt_h / ex_h)
    dxn = (dx - 0.0) / 0.1
    dyn_ = (dy - 0.0) / 0.1
    dwn = (dw - 0.0) / 0.2
    dhn = (dh - 0.0) / 0.2

    jr = lax.broadcasted_iota(jnp.int32, (_NROI, 1), 0)
    isfg = jr < _FG
    lab = jnp.where(isfg, glab, 0.0)
    clsf = jnp.round(lab)
    pidf = jnp.where(isfg, jnp.round(gpid), _BG_PID)

    cidx = lax.broadcasted_iota(jnp.int32, (_NROI, 4 * _NCLS), 1)
    cls_i = clsf.astype(jnp.int32)
    rel = cidx - 4 * cls_i
    fgm = cls_i > 0
    tvals = (jnp.where(rel == 0, dxn, 0.0) + jnp.where(rel == 1, dyn_, 0.0)
             + jnp.where(rel == 2, dwn, 0.0) + jnp.where(rel == 3, dhn, 0.0))
    tgt_ref[...] = jnp.where(fgm, tvals, 0.0)
    inw = jnp.where(fgm & (rel >= 0) & (rel <= 3), 1.0, 0.0)
    inw_ref[...] = inw
    outw_ref[...] = inw
    cls_ref[...] = clsf
    pid_ref[...] = pidf


def _run_tail(sel, gt):
    return pl.pallas_call(
        _tail_body,
        out_shape=[
            jax.ShapeDtypeStruct((_NROI, 1), jnp.float32),
            jax.ShapeDtypeStruct((_NROI, 4 * _NCLS), jnp.float32),
            jax.ShapeDtypeStruct((_NROI, 4 * _NCLS), jnp.float32),
            jax.ShapeDtypeStruct((_NROI, 4 * _NCLS), jnp.float32),
            jax.ShapeDtypeStruct((_NROI, 1), jnp.float32),
        ],
        in_specs=[
            pl.BlockSpec(memory_space=pltpu.VMEM),
            pl.BlockSpec(memory_space=pltpu.SMEM),
        ],
    )(sel, gt)


@jax.jit
def kernel(all_rois, gt_boxes):
    coords = jnp.concatenate([all_rois[:, 1:5], gt_boxes[:, :4]], axis=0)
    coords = jnp.pad(coords, ((0, _PAD - _NE), (0, 0)))
    x1 = coords[:, 0].reshape(_ROWS, _LANES)
    y1 = coords[:, 1].reshape(_ROWS, _LANES)
    x2 = coords[:, 2].reshape(_ROWS, _LANES)
    y2 = coords[:, 3].reshape(_ROWS, _LANES)
    mx, agf = _run_iou(x1, y1, x2, y2, gt_boxes)
    out = _run_sc(mx.reshape(_PAD), x1.reshape(_PAD), y1.reshape(_PAD),
                  x2.reshape(_PAD), y2.reshape(_PAD), agf.reshape(_PAD))
    sel = out[:_NROI, :8]
    clsf, tgt, inw, outw, pidf = _run_tail(sel, gt_boxes)
    rois = jnp.concatenate([jnp.zeros((_NROI, 1), jnp.float32), sel[:, :4]],
                           axis=1)
    clss = clsf[:, 0].astype(jnp.int32)
    pid = pidf[:, 0].astype(jnp.int32)
    return (rois, clss, tgt, inw, outw, pid)


# speedup vs baseline: 3.6373x; 1.1582x over previous
import jax
import jax.numpy as jnp
from jax import lax
from jax.experimental import pallas as pl
from jax.experimental.pallas import tpu as pltpu
from jax.experimental.pallas import tpu_sc as plsc

_N = 20000
_G = 64
_NE = _N + _G
_LANES = 128
_ROWS = 160
_PAD = _ROWS * _LANES
_NCLS = 81
_FG = 32
_NBG = 96
_NROI = 128
_BG_PID = 5532.0

_NC = 1
_NS = 16
_NW = _NC * _NS
_CHUNK = _PAD // _NW
_NVR = _CHUNK // 16
_OW = 16
_DUMMY = _NROI
_NSTG = 128


def _iou_body(x1_ref, y1_ref, x2_ref, y2_ref, gt_ref, mx_ref, ag_ref):
    x1 = x1_ref[...]
    y1 = y1_ref[...]
    x2 = x2_ref[...]
    y2 = y2_ref[...]
    area_b = (x2 - x1 + 1.0) * (y2 - y1 + 1.0)

    def gt_step(g, carry):
        mx, ag = carry
        gx1 = gt_ref[g, 0]
        gy1 = gt_ref[g, 1]
        gx2 = gt_ref[g, 2]
        gy2 = gt_ref[g, 3]
        area_q = (gx2 - gx1 + 1.0) * (gy2 - gy1 + 1.0)
        iw = jnp.minimum(x2, gx2) - jnp.maximum(x1, gx1) + 1.0
        ih = jnp.minimum(y2, gy2) - jnp.maximum(y1, gy1) + 1.0
        iw = jnp.maximum(iw, 0.0)
        ih = jnp.maximum(ih, 0.0)
        inter = iw * ih
        ua = area_b + area_q - inter
        iou = inter / ua
        upd = iou > mx
        mx = jnp.where(upd, iou, mx)
        ag = jnp.where(upd, g, ag)
        return mx, ag

    mx0 = jnp.full((_ROWS, _LANES), -1.0, jnp.float32)
    ag0 = jnp.zeros((_ROWS, _LANES), jnp.int32)
    mx, ag = lax.fori_loop(0, _G, gt_step, (mx0, ag0), unroll=16)
    mx_ref[...] = mx
    ag_ref[...] = ag.astype(jnp.float32)


def _run_iou(x1, y1, x2, y2, gt):
    return pl.pallas_call(
        _iou_body,
        out_shape=[
            jax.ShapeDtypeStruct((_ROWS, _LANES), jnp.float32),
            jax.ShapeDtypeStruct((_ROWS, _LANES), jnp.float32),
        ],
        in_specs=[
            pl.BlockSpec(memory_space=pltpu.VMEM),
            pl.BlockSpec(memory_space=pltpu.VMEM),
            pl.BlockSpec(memory_space=pltpu.VMEM),
            pl.BlockSpec(memory_space=pltpu.VMEM),
            pl.BlockSpec(memory_space=pltpu.SMEM),
        ],
    )(x1, y1, x2, y2, gt)


def _sc_body(mx_hbm, x1_hbm, y1_hbm, x2_hbm, y2_hbm, ag_hbm, out_hbm,
             mx_v, f0_v, f1_v, f2_v, f3_v, f4_v,
             candfg_v, candbg_v, cnt_v, allc_v, shared, stg_v, slot_v, sem):
    wid = lax.axis_index("s") * _NC + lax.axis_index("c")
    base = wid * _CHUNK
    fv = (f0_v, f1_v, f2_v, f3_v, f4_v)
    cps = [pltpu.async_copy(mx_hbm.at[pl.ds(base, _CHUNK)], mx_v, sem)]
    for c, src in enumerate((x1_hbm, y1_hbm, x2_hbm, y2_hbm, ag_hbm)):
        cps.append(pltpu.async_copy(src.at[pl.ds(base, _CHUNK)], fv[c], sem))
    for cp in cps:
        cp.wait()

    zeros16 = jnp.zeros((16,), jnp.int32)
    for i in range(_FG // 16 + 1):
        candfg_v[pl.ds(16 * i, 16)] = zeros16
    for i in range(_NBG // 16 + 1):
        candbg_v[pl.ds(16 * i, 16)] = zeros16

    lane = lax.iota(jnp.int32, 16)
    carry_fg = jnp.zeros((16,), jnp.int32)
    carry_bg = jnp.zeros((16,), jnp.int32)
    for v in range(_NVR):
        mxv = mx_v[pl.ds(16 * v, 16)]
        lin = base + 16 * v + lane
        loc = 16 * v + lane
        ok = lin < _NE
        mfg = (mxv >= 0.5) & ok
        mbg = (mxv < 0.5) & (mxv >= 0.0) & ok
        rfg = plsc.cumsum(mfg.astype(jnp.int32)) + carry_fg
        rbg = plsc.cumsum(mbg.astype(jnp.int32)) + carry_bg
        plsc.store_scatter(candfg_v, [rfg - 1], loc, mask=mfg & (rfg <= _FG))
        plsc.store_scatter(candbg_v, [rbg - 1], loc, mask=mbg & (rbg <= _NBG))
        carry_fg = carry_fg + plsc.all_reduce_population_count(mfg)
        carry_bg = carry_bg + plsc.all_reduce_population_count(mbg)

    cnt_v[...] = carry_fg * 4096 + carry_bg
    pltpu.sync_copy(cnt_v, shared.at[wid])
    plsc.subcore_barrier()
    pltpu.sync_copy(shared, allc_v)

    widv = jnp.full((16,), wid, jnp.int32)
    fg_off = jnp.zeros((16,), jnp.int32)
    bg_off = jnp.zeros((16,), jnp.int32)
    bg_tot = jnp.zeros((16,), jnp.int32)
    for w in range(_NW):
        rv = allc_v[w]
        fgw = rv // 4096
        bgw = rv % 4096
        before = jnp.full((16,), w, jnp.int32) < widv
        fg_off = fg_off + jnp.where(before, fgw, 0)
        bg_off = bg_off + jnp.where(before, bgw, 0)
        bg_tot = bg_tot + bgw

    def stage(i, loc, slot, valid):
        slot = jnp.where(valid, slot, _DUMMY)
        slot_v[pl.ds(16 * i, 16)] = slot
        row = 16 * i + lane
        for c in range(5):
            val = plsc.load_gather(fv[c], [loc])
            plsc.store_scatter(stg_v, [row, jnp.full((16,), c, jnp.int32)], val)

    for i in range(_FG // 16):
        k = 16 * i + lane
        slot = fg_off + k
        stage(i, candfg_v[pl.ds(16 * i, 16)], slot,
              (slot < _FG) & (k < carry_fg))
    iswid0 = widv == 0
    for i in range(_NBG // 16):
        k = 16 * i + lane
        slot = _FG + bg_off + k
        real = (slot < _NROI) & (k < carry_bg)
        fallb = iswid0 & (k >= bg_tot)
        loc = jnp.where(k < carry_bg, candbg_v[pl.ds(16 * i, 16)], 0)
        stage(2 + i, loc, slot, real | fallb)

    pltpu.async_copy(stg_v, out_hbm.at[slot_v], sem).wait()


def _run_sc(mx, x1, y1, x2, y2, ag):
    mesh = plsc.VectorSubcoreMesh(core_axis_name="c", subcore_axis_name="s",
                                  num_cores=_NC, num_subcores=_NS)
    f = pl.kernel(
        _sc_body,
        out_type=jax.ShapeDtypeStruct((_NROI + 8, _OW), jnp.float32),
        mesh=mesh,
        compiler_params=pltpu.CompilerParams(needs_layout_passes=False,
                                             use_tc_tiling_on_sc=False),
        scratch_types=[
            pltpu.VMEM((_CHUNK,), jnp.float32),
            pltpu.VMEM((_CHUNK,), jnp.float32),
            pltpu.VMEM((_CHUNK,), jnp.float32),
            pltpu.VMEM((_CHUNK,), jnp.float32),
            pltpu.VMEM((_CHUNK,), jnp.float32),
            pltpu.VMEM((_CHUNK,), jnp.float32),
            pltpu.VMEM((_FG + 16,), jnp.int32),
            pltpu.VMEM((_NBG + 16,), jnp.int32),
            pltpu.VMEM((16,), jnp.int32),
            pltpu.VMEM((_NW, 16), jnp.int32),
            pltpu.VMEM_SHARED((_NW, 16), jnp.int32),
            pltpu.VMEM((_NSTG, _OW), jnp.float32),
            pltpu.VMEM((_NSTG,), jnp.int32),
            pltpu.SemaphoreType.DMA,
        ],
    )
    return f(mx, x1, y1, x2, y2, ag)


def _tail_body(sel_ref, gt_ref, cls_ref, tgt_ref, inw_ref, outw_ref, pid_ref):
    sel = sel_ref[...]
    ex1 = sel[:, 0:1]
    ey1 = sel[:, 1:2]
    ex2 = sel[:, 2:3]
    ey2 = sel[:, 3:4]
    asg = sel[:, 4:5]

    col8 = lax.broadcasted_iota(jnp.int32, (_NROI, 8), 1)

    def ggt(g, c):
        gfv = g.astype(jnp.float32)
        m = asg == gfv
        vals = (jnp.where(col8 == 0, gt_ref[g, 0], 0.0)
                + jnp.where(col8 == 1, gt_ref[g, 1], 0.0)
                + jnp.where(col8 == 2, gt_ref[g, 2], 0.0)
                + jnp.where(col8 == 3, gt_ref[g, 3], 0.0)
                + jnp.where(col8 == 4, gt_ref[g, 4], 0.0)
                + jnp.where(col8 == 5, gt_ref[g, 5], 0.0))
        return c + jnp.where(m, vals, 0.0)

    gtacc = lax.fori_loop(0, _G, ggt, jnp.zeros((_NROI, 8), jnp.float32),
                          unroll=8)

    gx1 = gtacc[:, 0:1]
    gy1 = gtacc[:, 1:2]
    gx2 = gtacc[:, 2:3]
    gy2 = gtacc[:, 3:4]
    glab = gtacc[:, 4:5]
    gpid = gtacc[:, 5:6]

    ex_w = ex2 - ex1 + 1.0
    ex_h = ey2 - ey1 + 1.0
    ex_cx = ex1 + 0.5 * ex_w
    ex_cy = ey1 + 0.5 * ex_h
    gt_w = gx2 - gx1 + 1.0
    gt_h = gy2 - gy1 + 1.0
    gt_cx = gx1 + 0.5 * gt_w
    gt_cy = gy1 + 0.5 * gt_h
    dx = (gt_cx - ex_cx) / ex_w
    dy = (gt_cy - ex_cy) / ex_h
    dw = jnp.log(gt_w / ex_w)
    dh = jnp.log(gt_h / ex_h)
    dxn = (dx - 0.0) / 0.1
    dyn_ = (dy - 0.0) / 0.1
    dwn = (dw - 0.0) / 0.2
    dhn = (dh - 0.0) / 0.2

    jr = lax.broadcasted_iota(jnp.int32, (_NROI, 1), 0)
    isfg = jr < _FG
    lab = jnp.where(isfg, glab, 0.0)
    clsf = jnp.round(lab)
    pidf = jnp.where(isfg, jnp.round(gpid), _BG_PID)

    cidx = lax.broadcasted_iota(jnp.int32, (_NROI, 4 * _NCLS), 1)
    cls_i = clsf.astype(jnp.int32)
    rel = cidx - 4 * cls_i
    fgm = cls_i > 0
    tvals = (jnp.where(rel == 0, dxn, 0.0) + jnp.where(rel == 1, dyn_, 0.0)
             + jnp.where(rel == 2, dwn, 0.0) + jnp.where(rel == 3, dhn, 0.0))
    tgt_ref[...] = jnp.where(fgm, tvals, 0.0)
    inw = jnp.where(fgm & (rel >= 0) & (rel <= 3), 1.0, 0.0)
    inw_ref[...] = inw
    outw_ref[...] = inw
    cls_ref[...] = clsf
    pid_ref[...] = pidf


def _run_tail(sel, gt):
    return pl.pallas_call(
        _tail_body,
        out_shape=[
            jax.ShapeDtypeStruct((_NROI, 1), jnp.float32),
            jax.ShapeDtypeStruct((_NROI, 4 * _NCLS), jnp.float32),
            jax.ShapeDtypeStruct((_NROI, 4 * _NCLS), jnp.float32),
            jax.ShapeDtypeStruct((_NROI, 4 * _NCLS), jnp.float32),
            jax.ShapeDtypeStruct((_NROI, 1), jnp.float32),
        ],
        in_specs=[
            pl.BlockSpec(memory_space=pltpu.VMEM),
            pl.BlockSpec(memory_space=pltpu.SMEM),
        ],
    )(sel, gt)


@jax.jit
def kernel(all_rois, gt_boxes):
    coords = jnp.concatenate([all_rois[:, 1:5], gt_boxes[:, :4]], axis=0)
    coords = jnp.pad(coords, ((0, _PAD - _NE), (0, 0)))
    x1 = coords[:, 0].reshape(_ROWS, _LANES)
    y1 = coords[:, 1].reshape(_ROWS, _LANES)
    x2 = coords[:, 2].reshape(_ROWS, _LANES)
    y2 = coords[:, 3].reshape(_ROWS, _LANES)
    mx, agf = _run_iou(x1, y1, x2, y2, gt_boxes)
    out = _run_sc(mx.reshape(_PAD), x1.reshape(_PAD), y1.reshape(_PAD),
                  x2.reshape(_PAD), y2.reshape(_PAD), agf.reshape(_PAD))
    sel = out[:_NROI, :8]
    clsf, tgt, inw, outw, pidf = _run_tail(sel, gt_boxes)
    rois = jnp.concatenate([jnp.zeros((_NROI, 1), jnp.float32), sel[:, :4]],
                           axis=1)
    clss = clsf[:, 0].astype(jnp.int32)
    pid = pidf[:, 0].astype(jnp.int32)
    return (rois, clss, tgt, inw, outw, pid)
